# Initial kernel scaffold; baseline (speedup 1.0000x reference)
import jax
import jax.numpy as jnp
from jax.experimental import pallas as pl
from jax.experimental.pallas import tpu as pltpu

_NE = 1024
_D = 64
_N = 16384
_CC = 0.25
_BLK = 1024


def _vq_body(x_ref, e_ref, enc_ref, q_ref, loss_ref, ppl_ref, counts_ref, sse_ref):
    i = pl.program_id(0)
    x = x_ref[...]
    e = e_ref[...]

    x2 = jnp.sum(x * x, axis=1, keepdims=True)
    e2 = jnp.sum(e * e, axis=1)
    s = jax.lax.dot_general(x, e, (((1,), (1,)), ((), ())),
                            preferred_element_type=jnp.float32)
    dist = (x2 + e2[None, :]) - 2.0 * s
    idx = jnp.argmin(dist, axis=1)

    onehot = (jax.lax.broadcasted_iota(jnp.int32, (_BLK, _NE), 1)
              == idx[:, None]).astype(jnp.float32)
    enc_ref[...] = onehot

    q = jax.lax.dot_general(onehot, e, (((1,), (1,)), ((), ())),
                            preferred_element_type=jnp.float32)
    d = q - x
    q_ref[...] = x + d

    @pl.when(i == 0)
    def _init():
        counts_ref[...] = jnp.zeros_like(counts_ref)
        sse_ref[0] = 0.0

    counts_ref[...] += jnp.sum(onehot, axis=0, keepdims=True)
    sse_ref[0] += jnp.sum(d * d)

    loss_ref[0, 0] = 0.0
    ppl_ref[0, 0] = 0.0

    @pl.when(i == pl.num_programs(0) - 1)
    def _final():
        loss_ref[0, 0] = _CC * sse_ref[0] / (_N * _D)
        p = counts_ref[...] / _N
        ppl_ref[0, 0] = jnp.exp(-jnp.sum(p * jnp.log(p + 1e-10)))


def kernel(inputs, embedding_weight):
    grid = (_N // _BLK,)
    enc, q, loss, ppl = pl.pallas_call(
        _vq_body,
        grid=grid,
        in_specs=[
            pl.BlockSpec((_BLK, _D), lambda i: (i, 0)),
            pl.BlockSpec((_NE, _D), lambda i: (0, 0)),
        ],
        out_specs=[
            pl.BlockSpec((_BLK, _NE), lambda i: (i, 0)),
            pl.BlockSpec((_BLK, _D), lambda i: (i, 0)),
            pl.BlockSpec((1, 1), lambda i: (0, 0)),
            pl.BlockSpec((1, 1), lambda i: (0, 0)),
        ],
        out_shape=[
            jax.ShapeDtypeStruct((_N, _NE), jnp.float32),
            jax.ShapeDtypeStruct((_N, _D), jnp.float32),
            jax.ShapeDtypeStruct((1, 1), jnp.float32),
            jax.ShapeDtypeStruct((1, 1), jnp.float32),
        ],
        scratch_shapes=[
            pltpu.VMEM((1, _NE), jnp.float32),
            pltpu.SMEM((1,), jnp.float32),
        ],
    )(inputs, embedding_weight)
    return (loss[0, 0], q, ppl[0, 0], enc)


# speedup vs baseline: 3.2856x; 3.2856x over previous
import jax
import jax.numpy as jnp
from jax.experimental import pallas as pl
from jax.experimental.pallas import tpu as pltpu

_NE = 1024
_D = 64
_N = 16384
_CC = 0.25
_BLK = 1024


def _vq_body(x_ref, e_ref, enc_ref, q_ref, loss_ref, ppl_ref, counts_ref, sse_ref):
    i = pl.program_id(0)
    x = x_ref[...]
    e = e_ref[...]

    x2 = jnp.sum(x * x, axis=1, keepdims=True)
    e2 = jnp.sum(e * e, axis=1)
    s = jax.lax.dot_general(x, e, (((1,), (1,)), ((), ())),
                            preferred_element_type=jnp.float32)
    dist = (x2 + e2[None, :]) - 2.0 * s
    idx = jnp.argmin(dist, axis=1)

    onehot = (jax.lax.broadcasted_iota(jnp.int32, (_BLK, _NE), 1)
              == idx[:, None]).astype(jnp.float32)
    enc_ref[...] = onehot

    q = jax.lax.dot_general(onehot, e, (((1,), (0,)), ((), ())),
                            preferred_element_type=jnp.float32)
    d = q - x
    q_ref[...] = x + d

    @pl.when(i == 0)
    def _init():
        counts_ref[...] = jnp.zeros_like(counts_ref)
        sse_ref[0] = 0.0

    counts_ref[...] += jnp.sum(onehot, axis=0, keepdims=True)
    sse_ref[0] += jnp.sum(d * d)

    loss_ref[...] = jnp.zeros((1, 1), jnp.float32)
    ppl_ref[...] = jnp.zeros((1, 1), jnp.float32)

    @pl.when(i == pl.num_programs(0) - 1)
    def _final():
        loss_ref[...] = jnp.full((1, 1), _CC * sse_ref[0] / (_N * _D))
        p = counts_ref[...] / _N
        ppl_ref[...] = jnp.exp(-jnp.sum(p * jnp.log(p + 1e-10),
                                        keepdims=True))


def kernel(inputs, embedding_weight):
    grid = (_N // _BLK,)
    enc, q, loss, ppl = pl.pallas_call(
        _vq_body,
        grid=grid,
        in_specs=[
            pl.BlockSpec((_BLK, _D), lambda i: (i, 0)),
            pl.BlockSpec((_NE, _D), lambda i: (0, 0)),
        ],
        out_specs=[
            pl.BlockSpec((_BLK, _NE), lambda i: (i, 0)),
            pl.BlockSpec((_BLK, _D), lambda i: (i, 0)),
            pl.BlockSpec((1, 1), lambda i: (0, 0)),
            pl.BlockSpec((1, 1), lambda i: (0, 0)),
        ],
        out_shape=[
            jax.ShapeDtypeStruct((_N, _NE), jnp.float32),
            jax.ShapeDtypeStruct((_N, _D), jnp.float32),
            jax.ShapeDtypeStruct((1, 1), jnp.float32),
            jax.ShapeDtypeStruct((1, 1), jnp.float32),
        ],
        scratch_shapes=[
            pltpu.VMEM((1, _NE), jnp.float32),
            pltpu.SMEM((1,), jnp.float32),
        ],
    )(inputs, embedding_weight)
    return (loss[0, 0], q, ppl[0, 0], enc)


# speedup vs baseline: 3.3231x; 1.0114x over previous
import jax
import jax.numpy as jnp
from jax.experimental import pallas as pl
from jax.experimental.pallas import tpu as pltpu

_NE = 1024
_D = 64
_N = 16384
_CC = 0.25
_BLK = 1024


def _vq_body(x_ref, e_ref, enc_ref, q_ref, loss_ref, ppl_ref, counts_ref, sse_ref):
    i = pl.program_id(0)
    x = x_ref[...]
    e = e_ref[...]

    e2 = jnp.sum(e * e, axis=1)
    s = jax.lax.dot_general(x, e, (((1,), (1,)), ((), ())),
                            preferred_element_type=jnp.float32)
    score = 2.0 * s - e2[None, :]
    idx = jnp.argmax(score, axis=1)
    onehot = (jax.lax.broadcasted_iota(jnp.int32, (_BLK, _NE), 1)
              == idx[:, None]).astype(jnp.float32)
    enc_ref[...] = onehot

    q = jax.lax.dot_general(onehot, e, (((1,), (0,)), ((), ())),
                            preferred_element_type=jnp.float32)
    d = q - x
    q_ref[...] = x + d

    @pl.when(i == 0)
    def _init():
        counts_ref[...] = jnp.zeros_like(counts_ref)
        sse_ref[0] = 0.0

    counts_ref[...] += jnp.sum(onehot, axis=0, keepdims=True)
    sse_ref[0] += jnp.sum(d * d)

    loss_ref[...] = jnp.zeros((1, 1), jnp.float32)
    ppl_ref[...] = jnp.zeros((1, 1), jnp.float32)

    @pl.when(i == pl.num_programs(0) - 1)
    def _final():
        loss_ref[...] = jnp.full((1, 1), _CC * sse_ref[0] / (_N * _D))
        p = counts_ref[...] / _N
        ppl_ref[...] = jnp.exp(-jnp.sum(p * jnp.log(p + 1e-10),
                                        keepdims=True))


def kernel(inputs, embedding_weight):
    grid = (_N // _BLK,)
    enc, q, loss, ppl = pl.pallas_call(
        _vq_body,
        grid=grid,
        in_specs=[
            pl.BlockSpec((_BLK, _D), lambda i: (i, 0)),
            pl.BlockSpec((_NE, _D), lambda i: (0, 0)),
        ],
        out_specs=[
            pl.BlockSpec((_BLK, _NE), lambda i: (i, 0)),
            pl.BlockSpec((_BLK, _D), lambda i: (i, 0)),
            pl.BlockSpec((1, 1), lambda i: (0, 0)),
            pl.BlockSpec((1, 1), lambda i: (0, 0)),
        ],
        out_shape=[
            jax.ShapeDtypeStruct((_N, _NE), jnp.float32),
            jax.ShapeDtypeStruct((_N, _D), jnp.float32),
            jax.ShapeDtypeStruct((1, 1), jnp.float32),
            jax.ShapeDtypeStruct((1, 1), jnp.float32),
        ],
        scratch_shapes=[
            pltpu.VMEM((1, _NE), jnp.float32),
            pltpu.SMEM((1,), jnp.float32),
        ],
    )(inputs, embedding_weight)
    return (loss[0, 0], q, ppl[0, 0], enc)
